# initial kernel scaffold (unmeasured)
import jax
import jax.numpy as jnp
from jax import lax
from jax.experimental import pallas as pl
from jax.experimental.pallas import tpu as pltpu


def kernel(
    x,
):
    def body(*refs):
        pass

    out_shape = jax.ShapeDtypeStruct(..., jnp.float32)
    return pl.pallas_call(body, out_shape=out_shape)(...)


# baseline (device time: 11900 ns/iter reference)
import functools

import jax
import jax.numpy as jnp
from jax import lax
from jax.experimental import pallas as pl
from jax.experimental.pallas import tpu as pltpu

N_DEV = 4


def kernel(x):
    m, n = x.shape

    def body(x_ref, out_ref, total_ref, recv_ref, send_sems, recv_sems):
        my = lax.axis_index("i")

        acc = x_ref[:, :]
        s = 1
        while s < m:
            acc = acc * jnp.concatenate(
                [jnp.ones((s, n), acc.dtype), acc[: m - s, :]], axis=0
            )
            s *= 2

        total_ref[:, :] = acc[m - 1 : m, :]
        for j in range(N_DEV - 1):
            recv_ref[j, :, :] = jnp.ones((1, n), acc.dtype)

        barrier = pltpu.get_barrier_semaphore()
        for d in range(N_DEV):

            @pl.when(my != d)
            def _signal(d=d):
                pl.semaphore_signal(
                    barrier,
                    inc=1,
                    device_id=(d,),
                    device_id_type=pl.DeviceIdType.MESH,
                )

        pl.semaphore_wait(barrier, N_DEV - 1)

        for j in range(N_DEV):
            for k in range(j + 1, N_DEV):

                @pl.when(my == j)
                def _send(j=j, k=k):
                    pltpu.make_async_remote_copy(
                        src_ref=total_ref,
                        dst_ref=recv_ref.at[j],
                        send_sem=send_sems.at[k - j - 1],
                        recv_sem=recv_sems.at[j],
                        device_id=(k,),
                        device_id_type=pl.DeviceIdType.MESH,
                    ).start()

        for j in range(N_DEV - 1):

            @pl.when(j < my)
            def _recv(j=j):
                pltpu.make_async_remote_copy(
                    src_ref=total_ref,
                    dst_ref=recv_ref.at[j],
                    send_sem=send_sems.at[0],
                    recv_sem=recv_sems.at[j],
                    device_id=(j,),
                    device_id_type=pl.DeviceIdType.MESH,
                ).wait_recv()

        prefix = recv_ref[0, :, :]
        for j in range(1, N_DEV - 1):
            prefix = prefix * recv_ref[j, :, :]

        out_ref[:, :] = acc * prefix

        for j in range(N_DEV):
            for k in range(j + 1, N_DEV):

                @pl.when(my == j)
                def _drain(j=j, k=k):
                    pltpu.make_async_remote_copy(
                        src_ref=total_ref,
                        dst_ref=recv_ref.at[j],
                        send_sem=send_sems.at[k - j - 1],
                        recv_sem=recv_sems.at[j],
                        device_id=(k,),
                        device_id_type=pl.DeviceIdType.MESH,
                    ).wait_send()

        @functools.partial(pl.run_scoped, sem=pltpu.SemaphoreType.REGULAR)
        def _exit_barrier(sem):
            for d in range(N_DEV):

                @pl.when(my != d)
                def _signal(d=d):
                    pl.semaphore_signal(
                        sem,
                        inc=1,
                        device_id=(d,),
                        device_id_type=pl.DeviceIdType.MESH,
                    )

            pl.semaphore_wait(sem, N_DEV - 1)

    return pl.pallas_call(
        body,
        out_shape=jax.ShapeDtypeStruct((m, n), x.dtype),
        in_specs=[pl.BlockSpec(memory_space=pltpu.VMEM)],
        out_specs=pl.BlockSpec(memory_space=pltpu.VMEM),
        scratch_shapes=[
            pltpu.VMEM((1, n), x.dtype),
            pltpu.VMEM((N_DEV - 1, 1, n), x.dtype),
            pltpu.SemaphoreType.DMA((N_DEV - 1,)),
            pltpu.SemaphoreType.DMA((N_DEV - 1,)),
        ],
        compiler_params=pltpu.CompilerParams(collective_id=0),
    )(x)


# device time: 9924 ns/iter; 1.1991x vs baseline; 1.1991x over previous
import functools

import jax
import jax.numpy as jnp
from jax import lax
from jax.experimental import pallas as pl
from jax.experimental.pallas import tpu as pltpu

N_DEV = 4


def kernel(x):
    m, n = x.shape

    def body(x_ref, out_ref, total_ref, recv_ref, send_sems, recv_sems):
        my = lax.axis_index("i")

        for j in range(N_DEV - 1):
            recv_ref[j, :, :] = jnp.ones((1, n), x_ref.dtype)

        p = x_ref[:, :]
        h = m
        while h > 1:
            h //= 2
            p = p[:h, :] * p[h:, :]
        total_ref[:, :] = p

        barrier = pltpu.get_barrier_semaphore()
        for d in range(N_DEV):

            @pl.when(my != d)
            def _signal(d=d):
                pl.semaphore_signal(
                    barrier,
                    inc=1,
                    device_id=(d,),
                    device_id_type=pl.DeviceIdType.MESH,
                )

        pl.semaphore_wait(barrier, N_DEV - 1)

        for j in range(N_DEV):
            for k in range(j + 1, N_DEV):

                @pl.when(my == j)
                def _send(j=j, k=k):
                    pltpu.make_async_remote_copy(
                        src_ref=total_ref,
                        dst_ref=recv_ref.at[j],
                        send_sem=send_sems.at[k - j - 1],
                        recv_sem=recv_sems.at[j],
                        device_id=(k,),
                        device_id_type=pl.DeviceIdType.MESH,
                    ).start()

        acc = x_ref[:, :]
        s = 1
        while s < m // 2:
            acc = acc * jnp.concatenate(
                [jnp.ones((s, n), acc.dtype), acc[: m - s, :]], axis=0
            )
            s *= 2

        for j in range(N_DEV - 1):

            @pl.when(j < my)
            def _recv(j=j):
                pltpu.make_async_remote_copy(
                    src_ref=total_ref,
                    dst_ref=recv_ref.at[j],
                    send_sem=send_sems.at[0],
                    recv_sem=recv_sems.at[j],
                    device_id=(j,),
                    device_id_type=pl.DeviceIdType.MESH,
                ).wait_recv()

        prefix = recv_ref[0, :, :]
        for j in range(1, N_DEV - 1):
            prefix = prefix * recv_ref[j, :, :]

        @functools.partial(pl.run_scoped, sem=pltpu.SemaphoreType.REGULAR)
        def _exit_barrier(sem):
            for d in range(N_DEV):

                @pl.when(my != d)
                def _signal(d=d):
                    pl.semaphore_signal(
                        sem,
                        inc=1,
                        device_id=(d,),
                        device_id_type=pl.DeviceIdType.MESH,
                    )

            half = m // 2
            shifted = jnp.concatenate(
                [
                    prefix * jnp.ones((half, n), acc.dtype),
                    acc[:half, :] * prefix,
                ],
                axis=0,
            )
            out_ref[:, :] = acc * shifted

            for j in range(N_DEV):
                for k in range(j + 1, N_DEV):

                    @pl.when(my == j)
                    def _drain(j=j, k=k):
                        pltpu.make_async_remote_copy(
                            src_ref=total_ref,
                            dst_ref=recv_ref.at[j],
                            send_sem=send_sems.at[k - j - 1],
                            recv_sem=recv_sems.at[j],
                            device_id=(k,),
                            device_id_type=pl.DeviceIdType.MESH,
                        ).wait_send()

            pl.semaphore_wait(sem, N_DEV - 1)

    return pl.pallas_call(
        body,
        out_shape=jax.ShapeDtypeStruct((m, n), x.dtype),
        in_specs=[pl.BlockSpec(memory_space=pltpu.VMEM)],
        out_specs=pl.BlockSpec(memory_space=pltpu.VMEM),
        scratch_shapes=[
            pltpu.VMEM((1, n), x.dtype),
            pltpu.VMEM((N_DEV - 1, 1, n), x.dtype),
            pltpu.SemaphoreType.DMA((N_DEV - 1,)),
            pltpu.SemaphoreType.DMA((N_DEV - 1,)),
        ],
        compiler_params=pltpu.CompilerParams(collective_id=0),
    )(x)


# device time: 9478 ns/iter; 1.2555x vs baseline; 1.0471x over previous
import functools

import jax
import jax.numpy as jnp
from jax import lax
from jax.experimental import pallas as pl
from jax.experimental.pallas import tpu as pltpu

N_DEV = 4


def _tile_scan(blk, r):
    b, _, n = blk.shape
    s = 1
    while s < r:
        blk = blk * jnp.concatenate(
            [jnp.ones((b, s, n), blk.dtype), blk[:, : r - s, :]], axis=1
        )
        s *= 2
    return blk


def kernel(x):
    m, n = x.shape

    def body(x_ref, out_ref, total_ref, recv_ref, send_sems, recv_sems):
        my = lax.axis_index("i")

        for j in range(N_DEV - 1):
            recv_ref[j, :, :] = jnp.ones((1, n), x_ref.dtype)

        p = x_ref[:, :]
        h = m
        while h > 1:
            h //= 2
            p = p[:h, :] * p[h:, :]
        total_ref[:, :] = p

        barrier = pltpu.get_barrier_semaphore()
        for d in range(N_DEV):

            @pl.when(my != d)
            def _signal(d=d):
                pl.semaphore_signal(
                    barrier,
                    inc=1,
                    device_id=(d,),
                    device_id_type=pl.DeviceIdType.MESH,
                )

        pl.semaphore_wait(barrier, N_DEV - 1)

        for j in range(N_DEV):
            for k in range(j + 1, N_DEV):

                @pl.when(my == j)
                def _send(j=j, k=k):
                    pltpu.make_async_remote_copy(
                        src_ref=total_ref,
                        dst_ref=recv_ref.at[j],
                        send_sem=send_sems.at[k - j - 1],
                        recv_sem=recv_sems.at[j],
                        device_id=(k,),
                        device_id_type=pl.DeviceIdType.MESH,
                    ).start()

        blk = x_ref[:, :].reshape(m // 8, 8, n)
        blk = _tile_scan(blk, 8)
        btot = blk[:, 7, :]

        b2 = _tile_scan(btot.reshape(m // 64, 8, n), 8)
        ctot = b2[:, 7, :]
        s = 1
        while s < m // 64:
            ctot = ctot * jnp.concatenate(
                [jnp.ones((s, n), ctot.dtype), ctot[: m // 64 - s, :]], axis=0
            )
            s *= 2
        cexc = jnp.concatenate(
            [jnp.ones((1, n), ctot.dtype), ctot[: m // 64 - 1, :]], axis=0
        )
        binc = (b2 * cexc[:, None, :]).reshape(m // 8, n)

        for j in range(N_DEV - 1):

            @pl.when(j < my)
            def _recv(j=j):
                pltpu.make_async_remote_copy(
                    src_ref=total_ref,
                    dst_ref=recv_ref.at[j],
                    send_sem=send_sems.at[0],
                    recv_sem=recv_sems.at[j],
                    device_id=(j,),
                    device_id_type=pl.DeviceIdType.MESH,
                ).wait_recv()

        prefix = recv_ref[0, :, :]
        for j in range(1, N_DEV - 1):
            prefix = prefix * recv_ref[j, :, :]

        @functools.partial(pl.run_scoped, sem=pltpu.SemaphoreType.REGULAR)
        def _exit_barrier(sem):
            for d in range(N_DEV):

                @pl.when(my != d)
                def _signal(d=d):
                    pl.semaphore_signal(
                        sem,
                        inc=1,
                        device_id=(d,),
                        device_id_type=pl.DeviceIdType.MESH,
                    )

            bexc = jnp.concatenate(
                [prefix, binc[: m // 8 - 1, :] * prefix], axis=0
            )

            out_ref[:, :] = (blk * bexc[:, None, :]).reshape(m, n)

            for j in range(N_DEV):
                for k in range(j + 1, N_DEV):

                    @pl.when(my == j)
                    def _drain(j=j, k=k):
                        pltpu.make_async_remote_copy(
                            src_ref=total_ref,
                            dst_ref=recv_ref.at[j],
                            send_sem=send_sems.at[k - j - 1],
                            recv_sem=recv_sems.at[j],
                            device_id=(k,),
                            device_id_type=pl.DeviceIdType.MESH,
                        ).wait_send()

            pl.semaphore_wait(sem, N_DEV - 1)

    return pl.pallas_call(
        body,
        out_shape=jax.ShapeDtypeStruct((m, n), x.dtype),
        in_specs=[pl.BlockSpec(memory_space=pltpu.VMEM)],
        out_specs=pl.BlockSpec(memory_space=pltpu.VMEM),
        scratch_shapes=[
            pltpu.VMEM((1, n), x.dtype),
            pltpu.VMEM((N_DEV - 1, 1, n), x.dtype),
            pltpu.SemaphoreType.DMA((N_DEV - 1,)),
            pltpu.SemaphoreType.DMA((N_DEV - 1,)),
        ],
        compiler_params=pltpu.CompilerParams(collective_id=0),
    )(x)


# device time: 8552 ns/iter; 1.3915x vs baseline; 1.1083x over previous
import jax
import jax.numpy as jnp
from jax import lax
from jax.experimental import pallas as pl
from jax.experimental.pallas import tpu as pltpu

N_DEV = 4


def kernel(x):
    m, n = x.shape

    def body(x_ref, out_ref, total_ref, recv_ref, send_sems, recv_sems):
        my = lax.axis_index("i")

        barrier = pltpu.get_barrier_semaphore()
        for j in range(N_DEV - 1):
            for k in range(j + 1, N_DEV):

                @pl.when(my == k)
                def _credit(j=j):
                    pl.semaphore_signal(
                        barrier,
                        inc=1,
                        device_id=(j,),
                        device_id_type=pl.DeviceIdType.MESH,
                    )

        p = x_ref[:, :]
        h = m
        while h > 1:
            h //= 2
            p = p[:h, :] * p[h:, :]
        total_ref[:, :] = p

        for j in range(N_DEV - 1):

            @pl.when(my == j)
            def _wait(j=j):
                pl.semaphore_wait(barrier, N_DEV - 1 - j)

        for j in range(N_DEV):
            for k in range(j + 1, N_DEV):

                @pl.when(my == j)
                def _send(j=j, k=k):
                    pltpu.make_async_remote_copy(
                        src_ref=total_ref,
                        dst_ref=recv_ref.at[j],
                        send_sem=send_sems.at[k - j - 1],
                        recv_sem=recv_sems.at[j],
                        device_id=(k,),
                        device_id_type=pl.DeviceIdType.MESH,
                    ).start()

        acc = x_ref[:, :]
        s = 1
        while s < m // 2:
            acc = acc * jnp.concatenate(
                [jnp.ones((s, n), acc.dtype), acc[: m - s, :]], axis=0
            )
            s *= 2

        for j in range(N_DEV - 1):

            @pl.when(j < my)
            def _recv(j=j):
                pltpu.make_async_remote_copy(
                    src_ref=total_ref,
                    dst_ref=recv_ref.at[j],
                    send_sem=send_sems.at[0],
                    recv_sem=recv_sems.at[j],
                    device_id=(j,),
                    device_id_type=pl.DeviceIdType.MESH,
                ).wait_recv()

        prefix = jnp.ones((1, n), acc.dtype)
        for j in range(N_DEV - 1):
            prefix = jnp.where(j < my, prefix * recv_ref[j, :, :], prefix)

        half = m // 2
        shifted = jnp.concatenate(
            [prefix * jnp.ones((half, n), acc.dtype), acc[:half, :] * prefix],
            axis=0,
        )
        out_ref[:, :] = acc * shifted

        for j in range(N_DEV):
            for k in range(j + 1, N_DEV):

                @pl.when(my == j)
                def _drain(j=j, k=k):
                    pltpu.make_async_remote_copy(
                        src_ref=total_ref,
                        dst_ref=recv_ref.at[j],
                        send_sem=send_sems.at[k - j - 1],
                        recv_sem=recv_sems.at[j],
                        device_id=(k,),
                        device_id_type=pl.DeviceIdType.MESH,
                    ).wait_send()

    return pl.pallas_call(
        body,
        out_shape=jax.ShapeDtypeStruct((m, n), x.dtype),
        in_specs=[pl.BlockSpec(memory_space=pltpu.VMEM)],
        out_specs=pl.BlockSpec(memory_space=pltpu.VMEM),
        scratch_shapes=[
            pltpu.VMEM((1, n), x.dtype),
            pltpu.VMEM((N_DEV - 1, 1, n), x.dtype),
            pltpu.SemaphoreType.DMA((N_DEV - 1,)),
            pltpu.SemaphoreType.DMA((N_DEV - 1,)),
        ],
        compiler_params=pltpu.CompilerParams(collective_id=0),
    )(x)


# device time: 7329 ns/iter; 1.6237x vs baseline; 1.1669x over previous
import jax
import jax.numpy as jnp
from jax import lax
from jax.experimental import pallas as pl
from jax.experimental.pallas import tpu as pltpu

N_DEV = 4


def kernel(x):
    m, n = x.shape

    def body(x_ref, out_ref, total_ref, recv_ref, send_sems, recv_sems):
        my = lax.axis_index("i")

        barrier = pltpu.get_barrier_semaphore()
        for j in range(N_DEV - 1):
            for k in range(j + 1, N_DEV):

                @pl.when(my == k)
                def _credit(j=j):
                    pl.semaphore_signal(
                        barrier,
                        inc=1,
                        device_id=(j,),
                        device_id_type=pl.DeviceIdType.MESH,
                    )

        p = x_ref[:, :]
        h = m
        while h > 1:
            h //= 2
            p = p[:h, :] * p[h:, :]
        total_ref[:, :] = p

        acc = x_ref[:, :]
        s = 1
        while s < m // 2:
            acc = acc * jnp.concatenate(
                [jnp.ones((s, n), acc.dtype), acc[: m - s, :]], axis=0
            )
            s *= 2

        for j in range(N_DEV - 1):

            @pl.when(my == j)
            def _wait(j=j):
                pl.semaphore_wait(barrier, N_DEV - 1 - j)

        for j in range(N_DEV):
            for k in range(j + 1, N_DEV):

                @pl.when(my == j)
                def _send(j=j, k=k):
                    pltpu.make_async_remote_copy(
                        src_ref=total_ref,
                        dst_ref=recv_ref.at[j],
                        send_sem=send_sems.at[k - j - 1],
                        recv_sem=recv_sems.at[j],
                        device_id=(k,),
                        device_id_type=pl.DeviceIdType.MESH,
                    ).start()

        for j in range(N_DEV - 1):

            @pl.when(j < my)
            def _recv(j=j):
                pltpu.make_async_remote_copy(
                    src_ref=total_ref,
                    dst_ref=recv_ref.at[j],
                    send_sem=send_sems.at[0],
                    recv_sem=recv_sems.at[j],
                    device_id=(j,),
                    device_id_type=pl.DeviceIdType.MESH,
                ).wait_recv()

        prefix = jnp.ones((1, n), acc.dtype)
        for j in range(N_DEV - 1):
            prefix = jnp.where(j < my, prefix * recv_ref[j, :, :], prefix)

        half = m // 2
        shifted = jnp.concatenate(
            [prefix * jnp.ones((half, n), acc.dtype), acc[:half, :] * prefix],
            axis=0,
        )
        out_ref[:, :] = acc * shifted

        for j in range(N_DEV):
            for k in range(j + 1, N_DEV):

                @pl.when(my == j)
                def _drain(j=j, k=k):
                    pltpu.make_async_remote_copy(
                        src_ref=total_ref,
                        dst_ref=recv_ref.at[j],
                        send_sem=send_sems.at[k - j - 1],
                        recv_sem=recv_sems.at[j],
                        device_id=(k,),
                        device_id_type=pl.DeviceIdType.MESH,
                    ).wait_send()

    return pl.pallas_call(
        body,
        out_shape=jax.ShapeDtypeStruct((m, n), x.dtype),
        in_specs=[pl.BlockSpec(memory_space=pltpu.VMEM)],
        out_specs=pl.BlockSpec(memory_space=pltpu.VMEM),
        scratch_shapes=[
            pltpu.VMEM((1, n), x.dtype),
            pltpu.VMEM((N_DEV - 1, 1, n), x.dtype),
            pltpu.SemaphoreType.DMA((N_DEV - 1,)),
            pltpu.SemaphoreType.DMA((N_DEV - 1,)),
        ],
        compiler_params=pltpu.CompilerParams(collective_id=0),
    )(x)
